# baseline (speedup 1.0000x reference)
import functools

import jax
import jax.numpy as jnp
from jax import lax
from jax.experimental import pallas as pl
from jax.experimental.pallas import tpu as pltpu
from jax.experimental.pallas import tpu_sc as plsc

HOPS = 3


def _make_segsum(bsz: int, mem: int, seg_len: int, d: int):
    info = plsc.get_sparse_core_info()
    nc, ns = info.num_cores, info.num_subcores
    nw = nc * ns
    n_mem_seg = bsz * mem
    assert n_mem_seg % nw == 0 and bsz % nw == 0
    mseg_w = n_mem_seg // nw
    useg_w = bsz // nw
    b_per_w = bsz // nw
    assert mseg_w == b_per_w * mem
    cs = 32
    assert mseg_w % cs == 0 and useg_w == cs
    nch_m = mseg_w // cs
    nch = nch_m + 1
    rpc = cs * seg_len
    assert rpc % 128 == 0
    ndma = rpc // 128
    assert d % 16 == 0
    dv = d // 16

    mesh = plsc.VectorSubcoreMesh(core_axis_name="c", subcore_axis_name="s")

    @functools.partial(
        pl.kernel,
        mesh=mesh,
        out_type=[
            jax.ShapeDtypeStruct((bsz, mem, d), jnp.float32),
            jax.ShapeDtypeStruct((bsz, d), jnp.float32),
        ],
        compiler_params=pltpu.CompilerParams(use_tc_tiling_on_sc=False),
        scratch_types=[
            pltpu.VMEM((2 * rpc,), jnp.int32),
            pltpu.VMEM((2 * rpc, d), jnp.float32),
            pltpu.VMEM((mseg_w + useg_w, d), jnp.float32),
            pltpu.SemaphoreType.DMA,
            pltpu.SemaphoreType.DMA,
        ],
    )
    def segsum(midx_hbm, uidx_hbm, emb_hbm, oms_hbm, octx_hbm,
               idx_v, rows_v, out_v, sem_i, sem_g):
        wid = lax.axis_index("s") * nc + lax.axis_index("c")
        mbase = wid * mseg_w * seg_len
        ubase = wid * useg_w * seg_len

        def fire_idx(c):
            buf = lax.rem(c, 2) * rpc
            dst = idx_v.at[pl.ds(buf, rpc)]

            @pl.when(c < nch_m)
            def _():
                pltpu.async_copy(
                    midx_hbm.at[pl.ds(mbase + c * rpc, rpc)], dst, sem_i
                )

            @pl.when(c >= nch_m)
            def _():
                pltpu.async_copy(uidx_hbm.at[pl.ds(ubase, rpc)], dst, sem_i)

        def wait_idx():
            pltpu.make_async_copy(
                uidx_hbm.at[pl.ds(0, rpc)], idx_v.at[pl.ds(0, rpc)], sem_i
            ).wait()

        def fire_gathers(c):
            buf = lax.rem(c, 2) * rpc
            for j in range(ndma):
                pltpu.async_copy(
                    emb_hbm.at[idx_v.at[pl.ds(buf + j * 128, 128)]],
                    rows_v.at[pl.ds(buf + j * 128, 128)],
                    sem_g,
                )

        def wait_gathers():
            for j in range(ndma):
                pltpu.make_async_copy(
                    emb_hbm.at[pl.ds(0, 128)],
                    rows_v.at[pl.ds(j * 128, 128)],
                    sem_g,
                ).wait()

        fire_idx(0)
        fire_idx(1)
        wait_idx()
        fire_gathers(0)

        def chunk_body(c, _):
            wait_idx()
            fire_gathers(jnp.minimum(c + 1, nch - 1))
            wait_gathers()
            fire_idx(jnp.minimum(c + 2, nch - 1))
            rbase = lax.rem(c, 2) * rpc

            def seg_body(s, _):
                base = rbase + s * seg_len
                for v in range(dv):
                    acc = rows_v[base, pl.ds(v * 16, 16)]
                    for j in range(1, seg_len):
                        acc = acc + rows_v[base + j, pl.ds(v * 16, 16)]
                    out_v[c * cs + s, pl.ds(v * 16, 16)] = acc
                return 0

            lax.fori_loop(0, cs, seg_body, 0)
            return 0

        lax.fori_loop(0, nch, chunk_body, 0)
        wait_idx()
        wait_gathers()

        def out_body(i, _):
            pltpu.async_copy(
                out_v.at[pl.ds(i * mem, mem)],
                oms_hbm.at[wid * b_per_w + i],
                sem_i,
            )
            return 0

        lax.fori_loop(0, b_per_w, out_body, 0)
        pltpu.sync_copy(
            out_v.at[pl.ds(mseg_w, useg_w)],
            octx_hbm.at[pl.ds(wid * useg_w, useg_w)],
        )

        def drain_body(i, _):
            pltpu.make_async_copy(
                oms_hbm.at[wid * b_per_w],
                out_v.at[pl.ds(0, mem)],
                sem_i,
            ).wait()
            return 0

        lax.fori_loop(0, b_per_w, drain_body, 0)

    return segsum


def _hops_body(ms_ref, ctx_ref, w_ref, b_ref, out_ref):
    mem = ms_ref.shape[1]
    cols = [ms_ref[:, m, :].T for m in range(mem)]
    ms = jnp.stack(cols, axis=0)
    ctx = ctx_ref[...].T
    w = w_ref[...]
    bv = b_ref[...]
    for _ in range(HOPS):
        attn = jnp.sum(ms * ctx[None, :, :], axis=1)
        attn = attn - jnp.max(attn, axis=0, keepdims=True)
        e = jnp.exp(attn)
        p = e / jnp.sum(e, axis=0, keepdims=True)
        stories = jnp.sum(p[:, None, :] * ms, axis=0)
        ctx = (
            jnp.dot(w, ctx, preferred_element_type=jnp.float32) + bv + stories
        )
    out_ref[...] = ctx.T


def kernel(utter, memory, emb, W, b):
    bsz, seq = utter.shape
    _, mem, _ = memory.shape
    _, d = emb.shape

    midx = memory.reshape(-1).astype(jnp.int32)
    uidx = utter.reshape(-1).astype(jnp.int32)

    ms2, ctx0 = _make_segsum(bsz, mem, seq, d)(midx, uidx, emb)

    bb = 128
    grid = (bsz // bb,)
    out = pl.pallas_call(
        _hops_body,
        grid=grid,
        in_specs=[
            pl.BlockSpec((bb, mem, d), lambda i: (i, 0, 0)),
            pl.BlockSpec((bb, d), lambda i: (i, 0)),
            pl.BlockSpec((d, d), lambda i: (0, 0)),
            pl.BlockSpec((d, 1), lambda i: (0, 0)),
        ],
        out_specs=pl.BlockSpec((bb, d), lambda i: (i, 0)),
        out_shape=jax.ShapeDtypeStruct((bsz, d), jnp.float32),
    )(ms2, ctx0, W, b.reshape(d, 1))
    return out
